# baseline (speedup 1.0000x reference)
import functools

import jax
import jax.numpy as jnp
from jax import lax
from jax.experimental import pallas as pl
from jax.experimental.pallas import tpu as pltpu
from jax.experimental.pallas import tpu_sc as plsc

_B, _C, _H, _W = 8, 16, 512, 512
_HB = 128
_NH = _H // _HB
_NBAND = 4
_BANDH = _H // _NBAND
_DELTA = 10.0
_NS_GT0 = 60.0
_NS_EQ0 = 10.0


def _sc_body(scr_hbm, scol_hbm, srow_hbm, blk, colp, rowp, sem):
    wid = lax.axis_index("s") * 2 + lax.axis_index("c")
    b = wid // _NBAND
    band = wid % _NBAND

    pltpu.sync_copy(scr_hbm.at[b, pl.ds(band * _BANDH, _BANDH)], blk)

    one = jnp.int32(1)
    zeros = jnp.zeros((16,), jnp.int32)
    io16 = lax.broadcasted_iota(jnp.int32, (16,), 0)

    for g in range(_BANDH // 16):
        rb = g * 16

        def chunk_step(k, racc, rb=rb, first=(g == 0)):
            vs = [one << blk[rb + j, pl.ds(k * 16, 16)] for j in range(16)]
            t = vs[0]
            for j in range(1, 16):
                t = t | vs[j]
            if first:
                colp[pl.ds(k * 16, 16)] = t
            else:
                colp[pl.ds(k * 16, 16)] = colp[pl.ds(k * 16, 16)] | t
            return tuple(racc[j] | vs[j] for j in range(16))

        racc = lax.fori_loop(0, _W // 16, chunk_step, (zeros,) * 16)

        out = zeros
        for j in range(16):
            v = racc[j]
            for sh in (1, 2, 4, 8):
                v = v | v.at[(io16 + sh) & 15].get(mode="promise_in_bounds")
            out = jnp.where(io16 == j, v, out)
        rowp[pl.ds(rb, 16)] = out

    pltpu.sync_copy(colp, scol_hbm.at[b, band])
    pltpu.sync_copy(rowp, srow_hbm.at[b, pl.ds(band * _BANDH, _BANDH)])


def _sc_proj(scr):
    mesh = plsc.VectorSubcoreMesh(core_axis_name="c", subcore_axis_name="s")
    f = functools.partial(
        pl.kernel,
        mesh=mesh,
        out_type=[
            jax.ShapeDtypeStruct((_B, _NBAND, _W), jnp.int32),
            jax.ShapeDtypeStruct((_B, _H), jnp.int32),
        ],
        scratch_types=[
            pltpu.VMEM((_BANDH, _W), jnp.int32),
            pltpu.VMEM((_W,), jnp.int32),
            pltpu.VMEM((_BANDH,), jnp.int32),
            pltpu.SemaphoreType.DMA,
        ],
    )(_sc_body)
    return f(scr)


def _lane_or(x):
    r = x[:, 0:128] | x[:, 128:256] | x[:, 256:384] | x[:, 384:512]
    for sh in (64, 32, 16, 8, 4, 2, 1):
        r = r | pltpu.roll(r, sh, axis=1)
    return r[:, 0:1]


def _sublane_or(x):
    r = x
    n = r.shape[0]
    while n > 8:
        n //= 2
        r = r[0:n] | r[n:2 * n]
    for sh in (4, 2, 1):
        r = r | pltpu.roll(r, sh, axis=0)
    return r[0:1]


def _transpose_col(col):
    io0 = lax.broadcasted_iota(jnp.int32, (_HB, _HB), 0)
    io1 = lax.broadcasted_iota(jnp.int32, (_HB, _HB), 1)
    eye = jnp.where(io0 == io1, 1.0, 0.0)
    lo = (col & 0xFF).astype(jnp.float32)
    hi = ((col >> 8) & 0xFF).astype(jnp.float32)
    dn = (((0,), (0,)), ((), ()))
    rlo = lax.dot_general(lo, eye, dn, preferred_element_type=jnp.float32)
    rhi = lax.dot_general(hi, eye, dn, preferred_element_type=jnp.float32)
    return rlo.astype(jnp.int32) | (rhi.astype(jnp.int32) << 8)


def _axis_stats(s, y, idx, limit):
    kd = dict(axis=1, keepdims=True)
    n1 = jnp.sum(s, **kd)
    d_pos = jnp.sum(s * (1.0 - y), **kd) / _NS_GT0
    d_neg = jnp.sum((1.0 - s) * y, **kd) / _NS_EQ0
    cs = jnp.minimum(1.0 / (1.0 + d_pos), 1.0 / (1.0 + d_neg))
    ext = jnp.floor(cs * _NS_EQ0)

    io0 = lax.broadcasted_iota(jnp.int32, (_W, _W), 0)
    io1 = lax.broadcasted_iota(jnp.int32, (_W, _W), 1)
    tri = jnp.where(io0 <= io1, 1.0, 0.0)
    cums = jnp.dot(s, tri, preferred_element_type=jnp.float32)

    so = n1 - cums
    sz = (float(_W - 1) - idx) - so
    one = jnp.float32(1.0)
    zero = jnp.float32(0.0)
    valid_bp = jnp.where(ext > 0.0, one, zero) * jnp.where(n1 > 0.0, one, zero)
    sel = jnp.where(
        s > 0.0,
        jnp.where(so < ext, one, zero),
        jnp.where(sz + n1 < ext, one, zero),
    )
    p = jnp.maximum(sel * valid_bp, y)

    any_p = jnp.max(p, **kd)
    lo = jnp.min(jnp.where(p > 0.0, idx, float(limit)), **kd)
    hi = jnp.max(jnp.where(p > 0.0, idx, -1.0), **kd)
    lo = jnp.maximum(0.0, lo - _DELTA)
    hi = jnp.minimum(float(limit - 1), hi + _DELTA)
    return any_p, lo, hi


def _main_kernel(out_ref, scol_ref, srow_ref, bbox_ref, pacc, rrow):
    h = pl.program_id(1)

    @pl.when(h == 0)
    def _init():
        pacc[...] = jnp.zeros_like(pacc)

    vals = out_ref[0]
    m = vals[0]
    for c in range(1, _C):
        m = jnp.maximum(m, vals[c])
    mb = jnp.zeros((_HB, _W), jnp.int32)
    for c in range(_C):
        mb = mb | jnp.where(vals[c] == m, jnp.int32(1 << c), 0)
    pm = mb & (-mb)

    pacc[...] = pacc[...] | pm
    rrow[0:1, pl.ds(h * _HB, _HB)] = _transpose_col(_lane_or(pm))

    @pl.when(h == _NH - 1)
    def _finish():
        pcol = _sublane_or(pacc[...])
        prow = rrow[0:1, :]
        scol = (scol_ref[0, 0:1] | scol_ref[0, 1:2]
                | scol_ref[0, 2:3] | scol_ref[0, 3:4])
        srow = srow_ref[0]

        csub = lax.broadcasted_iota(jnp.int32, (_C, 1), 0)

        def expand(packed):
            return ((jnp.broadcast_to(packed, (_C, _W)) >> csub) & 1).astype(jnp.float32)

        idx = lax.broadcasted_iota(jnp.int32, (1, _W), 1).astype(jnp.float32)
        anyx, x_min, x_max = _axis_stats(expand(scol), expand(pcol), idx, _W)
        anyy, y_min, y_max = _axis_stats(expand(srow), expand(prow), idx, _H)

        keep = anyx * anyy * jnp.where(csub != 0, 1.0, 0.0)
        bbox_ref[0] = jnp.concatenate([x_min, y_min, x_max, y_max], axis=1) * keep


@jax.jit
def _main(out, scol, srow):
    return pl.pallas_call(
        _main_kernel,
        grid=(_B, _NH),
        in_specs=[
            pl.BlockSpec((1, _C, _HB, _W), lambda b, h: (b, 0, h, 0)),
            pl.BlockSpec((1, _NBAND, _W), lambda b, h: (b, 0, 0)),
            pl.BlockSpec((1, 1, _W), lambda b, h: (b, 0, 0)),
        ],
        out_specs=pl.BlockSpec((1, _C, 4), lambda b, h: (b, 0, 0)),
        out_shape=jax.ShapeDtypeStruct((_B, _C, 4), jnp.float32),
        scratch_shapes=[
            pltpu.VMEM((_HB, _W), jnp.int32),
            pltpu.VMEM((1, _W), jnp.int32),
        ],
        compiler_params=pltpu.CompilerParams(
            dimension_semantics=("arbitrary", "arbitrary"),
        ),
    )(out, scol, srow)


def kernel(scribbles, outputs):
    scr = scribbles.astype(jnp.int32)
    scol, srow = _sc_proj(scr)
    bbox = _main(outputs, scol.reshape(_B, _NBAND, _W), srow.reshape(_B, 1, _H))
    return bbox


# speedup vs baseline: 1.2414x; 1.2414x over previous
import functools

import jax
import jax.numpy as jnp
from jax import lax
from jax.experimental import pallas as pl
from jax.experimental.pallas import tpu as pltpu
from jax.experimental.pallas import tpu_sc as plsc

_B, _C, _H, _W = 8, 16, 512, 512
_HB = 512
_NH = _H // _HB
_NBAND = 4
_BANDH = _H // _NBAND
_DELTA = 10.0
_NS_GT0 = 60.0
_NS_EQ0 = 10.0


def _sc_body(scr_hbm, scol_hbm, srow_hbm, blk, colp, rowp, sem):
    wid = lax.axis_index("s") * 2 + lax.axis_index("c")
    b = wid // _NBAND
    band = wid % _NBAND

    pltpu.sync_copy(scr_hbm.at[b, pl.ds(band * _BANDH, _BANDH)], blk)

    one = jnp.int32(1)
    zeros = jnp.zeros((16,), jnp.int32)
    io16 = lax.broadcasted_iota(jnp.int32, (16,), 0)

    for g in range(_BANDH // 16):
        rb = g * 16

        def chunk_step(k, racc, rb=rb, first=(g == 0)):
            vs = [one << blk[rb + j, pl.ds(k * 16, 16)] for j in range(16)]
            t = vs[0]
            for j in range(1, 16):
                t = t | vs[j]
            if first:
                colp[pl.ds(k * 16, 16)] = t
            else:
                colp[pl.ds(k * 16, 16)] = colp[pl.ds(k * 16, 16)] | t
            return tuple(racc[j] | vs[j] for j in range(16))

        racc = lax.fori_loop(0, _W // 16, chunk_step, (zeros,) * 16)

        out = zeros
        for j in range(16):
            v = racc[j]
            for sh in (1, 2, 4, 8):
                v = v | v.at[(io16 + sh) & 15].get(mode="promise_in_bounds")
            out = jnp.where(io16 == j, v, out)
        rowp[pl.ds(rb, 16)] = out

    pltpu.sync_copy(colp, scol_hbm.at[b, band])
    pltpu.sync_copy(rowp, srow_hbm.at[b, pl.ds(band * _BANDH, _BANDH)])


def _sc_proj(scr):
    mesh = plsc.VectorSubcoreMesh(core_axis_name="c", subcore_axis_name="s")
    f = functools.partial(
        pl.kernel,
        mesh=mesh,
        out_type=[
            jax.ShapeDtypeStruct((_B, _NBAND, _W), jnp.int32),
            jax.ShapeDtypeStruct((_B, _H), jnp.int32),
        ],
        scratch_types=[
            pltpu.VMEM((_BANDH, _W), jnp.int32),
            pltpu.VMEM((_W,), jnp.int32),
            pltpu.VMEM((_BANDH,), jnp.int32),
            pltpu.SemaphoreType.DMA,
        ],
    )(_sc_body)
    return f(scr)


def _lane_or(x):
    r = x[:, 0:128] | x[:, 128:256] | x[:, 256:384] | x[:, 384:512]
    for sh in (64, 32, 16, 8, 4, 2, 1):
        r = r | pltpu.roll(r, sh, axis=1)
    return r[:, 0:1]


def _sublane_or(x):
    r = x
    n = r.shape[0]
    while n > 8:
        n //= 2
        r = r[0:n] | r[n:2 * n]
    for sh in (4, 2, 1):
        r = r | pltpu.roll(r, sh, axis=0)
    return r[0:1]


def _pred_kernel(out_ref, pcol_ref, prow_ref):
    vals = out_ref[0]
    m = vals[0]
    for c in range(1, _C):
        m = jnp.maximum(m, vals[c])
    mb = jnp.zeros((_HB, _W), jnp.int32)
    for c in range(_C):
        mb = mb | jnp.where(vals[c] == m, jnp.int32(1 << c), 0)
    pm = mb & (-mb)

    pcol_ref[0] = _sublane_or(pm)
    prow_ref[0] = _lane_or(pm)


@jax.jit
def _pred_proj(out):
    return pl.pallas_call(
        _pred_kernel,
        grid=(_B,),
        in_specs=[pl.BlockSpec((1, _C, _HB, _W), lambda b: (b, 0, 0, 0))],
        out_specs=[
            pl.BlockSpec((1, 1, _W), lambda b: (b, 0, 0)),
            pl.BlockSpec((1, _H, 1), lambda b: (b, 0, 0)),
        ],
        out_shape=[
            jax.ShapeDtypeStruct((_B, 1, _W), jnp.int32),
            jax.ShapeDtypeStruct((_B, _H, 1), jnp.int32),
        ],
        compiler_params=pltpu.CompilerParams(
            dimension_semantics=("arbitrary",),
        ),
    )(out)


def _axis_stats(s, y, red_axis, idx, limit):
    kd = dict(axis=red_axis, keepdims=True)
    n1 = jnp.sum(s, **kd)
    d_pos = jnp.sum(s * (1.0 - y), **kd) / _NS_GT0
    d_neg = jnp.sum((1.0 - s) * y, **kd) / _NS_EQ0
    cs = jnp.minimum(1.0 / (1.0 + d_pos), 1.0 / (1.0 + d_neg))
    ext = jnp.floor(cs * _NS_EQ0)

    io0 = lax.broadcasted_iota(jnp.int32, (_W, _W), 0)
    io1 = lax.broadcasted_iota(jnp.int32, (_W, _W), 1)
    if red_axis == 1:
        tri = jnp.where(io0 <= io1, 1.0, 0.0)
        cums = jnp.dot(s, tri, preferred_element_type=jnp.float32)
    else:
        tri = jnp.where(io0 >= io1, 1.0, 0.0)
        cums = jnp.dot(tri, s, preferred_element_type=jnp.float32)

    so = n1 - cums
    sz = (float(_W - 1) - idx) - so
    one = jnp.float32(1.0)
    zero = jnp.float32(0.0)
    valid_bp = jnp.where(ext > 0.0, one, zero) * jnp.where(n1 > 0.0, one, zero)
    sel = jnp.where(
        s > 0.0,
        jnp.where(so < ext, one, zero),
        jnp.where(sz + n1 < ext, one, zero),
    )
    p = jnp.maximum(sel * valid_bp, y)

    any_p = jnp.max(p, **kd)
    lo = jnp.min(jnp.where(p > 0.0, idx, float(limit)), **kd)
    hi = jnp.max(jnp.where(p > 0.0, idx, -1.0), **kd)
    lo = jnp.maximum(0.0, lo - _DELTA)
    hi = jnp.minimum(float(limit - 1), hi + _DELTA)
    return any_p, lo, hi


def _diag_col(row):
    io0 = lax.broadcasted_iota(jnp.int32, (_C, _C), 0)
    io1 = lax.broadcasted_iota(jnp.int32, (_C, _C), 1)
    b = jnp.broadcast_to(row, (_C, _C))
    return jnp.sum(jnp.where(io0 == io1, b, 0.0), axis=1, keepdims=True)


def _fuse_kernel(pcol_ref, prow_ref, scol_ref, srow_ref, bbox_ref):
    scol = (scol_ref[0, 0:1] | scol_ref[0, 1:2]
            | scol_ref[0, 2:3] | scol_ref[0, 3:4])
    pcol = pcol_ref[0]
    prow = prow_ref[0]
    srow = srow_ref[0]

    csub = lax.broadcasted_iota(jnp.int32, (_C, 1), 0)
    clane = lax.broadcasted_iota(jnp.int32, (1, _C), 1)

    def expand_x(packed):
        return ((jnp.broadcast_to(packed, (_C, _W)) >> csub) & 1).astype(jnp.float32)

    def expand_y(packed):
        return ((jnp.broadcast_to(packed, (_H, _C)) >> clane) & 1).astype(jnp.float32)

    idx_x = lax.broadcasted_iota(jnp.int32, (1, _W), 1).astype(jnp.float32)
    anyx, x_min, x_max = _axis_stats(expand_x(scol), expand_x(pcol), 1, idx_x, _W)

    idx_y = lax.broadcasted_iota(jnp.int32, (_H, 1), 0).astype(jnp.float32)
    anyy_r, ylo_r, yhi_r = _axis_stats(expand_y(srow), expand_y(prow), 0, idx_y, _H)
    anyy = _diag_col(anyy_r)
    y_min = _diag_col(ylo_r)
    y_max = _diag_col(yhi_r)

    keep = anyx * anyy * jnp.where(csub != 0, 1.0, 0.0)
    bbox_ref[0] = jnp.concatenate([x_min, y_min, x_max, y_max], axis=1) * keep


@jax.jit
def _fuse(pcol, prow, scol, srow):
    return pl.pallas_call(
        _fuse_kernel,
        grid=(_B,),
        in_specs=[
            pl.BlockSpec((1, 1, _W), lambda b: (b, 0, 0)),
            pl.BlockSpec((1, _H, 1), lambda b: (b, 0, 0)),
            pl.BlockSpec((1, _NBAND, _W), lambda b: (b, 0, 0)),
            pl.BlockSpec((1, _H, 1), lambda b: (b, 0, 0)),
        ],
        out_specs=pl.BlockSpec((1, _C, 4), lambda b: (b, 0, 0)),
        out_shape=jax.ShapeDtypeStruct((_B, _C, 4), jnp.float32),
        compiler_params=pltpu.CompilerParams(
            dimension_semantics=("arbitrary",),
        ),
    )(pcol, prow, scol, srow)


def kernel(scribbles, outputs):
    scr = scribbles.astype(jnp.int32)
    scol, srow = _sc_proj(scr)
    pcol, prow = _pred_proj(outputs)
    bbox = _fuse(pcol, prow, scol.reshape(_B, _NBAND, _W),
                 srow.reshape(_B, _H, 1))
    return bbox


# speedup vs baseline: 1.2427x; 1.0010x over previous
import functools

import jax
import jax.numpy as jnp
from jax import lax
from jax.experimental import pallas as pl
from jax.experimental.pallas import tpu as pltpu
from jax.experimental.pallas import tpu_sc as plsc

_B, _C, _H, _W = 8, 16, 512, 512
_HB = 512
_NH = _H // _HB
_NBAND = 4
_BANDH = _H // _NBAND
_DELTA = 10.0
_NS_GT0 = 60.0
_NS_EQ0 = 10.0


_NCHUNK = 4
_CH = _BANDH // _NCHUNK


def _sc_body(scr_hbm, scol_hbm, srow_hbm, blk, colp, rowp, s0, s1, s2, s3):
    wid = lax.axis_index("s") * 2 + lax.axis_index("c")
    b = wid // _NBAND
    band = wid % _NBAND

    sems = (s0, s1, s2, s3)
    hnds = [
        pltpu.async_copy(
            scr_hbm.at[b, pl.ds(band * _BANDH + c * _CH, _CH)],
            blk.at[pl.ds(c * _CH, _CH)],
            sems[c],
        )
        for c in range(_NCHUNK)
    ]

    one = jnp.int32(1)
    zeros = jnp.zeros((16,), jnp.int32)
    io16 = lax.broadcasted_iota(jnp.int32, (16,), 0)

    for g in range(_BANDH // 16):
        rb = g * 16
        if rb % _CH == 0:
            hnds[rb // _CH].wait()

        def chunk_step(k, racc, rb=rb, first=(g == 0)):
            vs = [one << blk[rb + j, pl.ds(k * 16, 16)] for j in range(16)]
            t = vs[0]
            for j in range(1, 16):
                t = t | vs[j]
            if first:
                colp[pl.ds(k * 16, 16)] = t
            else:
                colp[pl.ds(k * 16, 16)] = colp[pl.ds(k * 16, 16)] | t
            return tuple(racc[j] | vs[j] for j in range(16))

        racc = lax.fori_loop(0, _W // 16, chunk_step, (zeros,) * 16)

        out = zeros
        for j in range(16):
            v = racc[j]
            for sh in (1, 2, 4, 8):
                v = v | v.at[(io16 + sh) & 15].get(mode="promise_in_bounds")
            out = jnp.where(io16 == j, v, out)
        rowp[pl.ds(rb, 16)] = out

    pltpu.sync_copy(colp, scol_hbm.at[b, band])
    pltpu.sync_copy(rowp, srow_hbm.at[b, pl.ds(band * _BANDH, _BANDH)])


def _sc_proj(scr):
    mesh = plsc.VectorSubcoreMesh(core_axis_name="c", subcore_axis_name="s")
    f = functools.partial(
        pl.kernel,
        mesh=mesh,
        out_type=[
            jax.ShapeDtypeStruct((_B, _NBAND, _W), jnp.int32),
            jax.ShapeDtypeStruct((_B, _H), jnp.int32),
        ],
        scratch_types=[
            pltpu.VMEM((_BANDH, _W), jnp.int32),
            pltpu.VMEM((_W,), jnp.int32),
            pltpu.VMEM((_BANDH,), jnp.int32),
            pltpu.SemaphoreType.DMA,
            pltpu.SemaphoreType.DMA,
            pltpu.SemaphoreType.DMA,
            pltpu.SemaphoreType.DMA,
        ],
    )(_sc_body)
    return f(scr)


def _lane_or(x):
    r = x[:, 0:128] | x[:, 128:256] | x[:, 256:384] | x[:, 384:512]
    for sh in (64, 32, 16, 8, 4, 2, 1):
        r = r | pltpu.roll(r, sh, axis=1)
    return r[:, 0:1]


def _sublane_or(x):
    r = x
    n = r.shape[0]
    while n > 8:
        n //= 2
        r = r[0:n] | r[n:2 * n]
    for sh in (4, 2, 1):
        r = r | pltpu.roll(r, sh, axis=0)
    return r[0:1]


def _pred_kernel(out_ref, pcol_ref, prow_ref):
    vals = out_ref[0]
    m = vals[0]
    for c in range(1, _C):
        m = jnp.maximum(m, vals[c])
    mb = jnp.zeros((_HB, _W), jnp.int32)
    for c in range(_C):
        mb = mb | jnp.where(vals[c] == m, jnp.int32(1 << c), 0)
    pm = mb & (-mb)

    pcol_ref[0] = _sublane_or(pm)
    prow_ref[0] = _lane_or(pm)


@jax.jit
def _pred_proj(out):
    return pl.pallas_call(
        _pred_kernel,
        grid=(_B,),
        in_specs=[pl.BlockSpec((1, _C, _HB, _W), lambda b: (b, 0, 0, 0))],
        out_specs=[
            pl.BlockSpec((1, 1, _W), lambda b: (b, 0, 0)),
            pl.BlockSpec((1, _H, 1), lambda b: (b, 0, 0)),
        ],
        out_shape=[
            jax.ShapeDtypeStruct((_B, 1, _W), jnp.int32),
            jax.ShapeDtypeStruct((_B, _H, 1), jnp.int32),
        ],
        compiler_params=pltpu.CompilerParams(
            dimension_semantics=("arbitrary",),
        ),
    )(out)


def _axis_stats(s, y, red_axis, idx, limit):
    kd = dict(axis=red_axis, keepdims=True)
    n1 = jnp.sum(s, **kd)
    d_pos = jnp.sum(s * (1.0 - y), **kd) / _NS_GT0
    d_neg = jnp.sum((1.0 - s) * y, **kd) / _NS_EQ0
    cs = jnp.minimum(1.0 / (1.0 + d_pos), 1.0 / (1.0 + d_neg))
    ext = jnp.floor(cs * _NS_EQ0)

    io0 = lax.broadcasted_iota(jnp.int32, (_W, _W), 0)
    io1 = lax.broadcasted_iota(jnp.int32, (_W, _W), 1)
    if red_axis == 1:
        tri = jnp.where(io0 <= io1, 1.0, 0.0)
        cums = jnp.dot(s, tri, preferred_element_type=jnp.float32)
    else:
        tri = jnp.where(io0 >= io1, 1.0, 0.0)
        cums = jnp.dot(tri, s, preferred_element_type=jnp.float32)

    so = n1 - cums
    sz = (float(_W - 1) - idx) - so
    one = jnp.float32(1.0)
    zero = jnp.float32(0.0)
    valid_bp = jnp.where(ext > 0.0, one, zero) * jnp.where(n1 > 0.0, one, zero)
    sel = jnp.where(
        s > 0.0,
        jnp.where(so < ext, one, zero),
        jnp.where(sz + n1 < ext, one, zero),
    )
    p = jnp.maximum(sel * valid_bp, y)

    any_p = jnp.max(p, **kd)
    lo = jnp.min(jnp.where(p > 0.0, idx, float(limit)), **kd)
    hi = jnp.max(jnp.where(p > 0.0, idx, -1.0), **kd)
    lo = jnp.maximum(0.0, lo - _DELTA)
    hi = jnp.minimum(float(limit - 1), hi + _DELTA)
    return any_p, lo, hi


def _diag_col(row):
    io0 = lax.broadcasted_iota(jnp.int32, (_C, _C), 0)
    io1 = lax.broadcasted_iota(jnp.int32, (_C, _C), 1)
    b = jnp.broadcast_to(row, (_C, _C))
    return jnp.sum(jnp.where(io0 == io1, b, 0.0), axis=1, keepdims=True)


def _fuse_kernel(pcol_ref, prow_ref, scol_ref, srow_ref, bbox_ref):
    scol = (scol_ref[0, 0:1] | scol_ref[0, 1:2]
            | scol_ref[0, 2:3] | scol_ref[0, 3:4])
    pcol = pcol_ref[0]
    prow = prow_ref[0]
    srow = srow_ref[0]

    csub = lax.broadcasted_iota(jnp.int32, (_C, 1), 0)
    clane = lax.broadcasted_iota(jnp.int32, (1, _C), 1)

    def expand_x(packed):
        return ((jnp.broadcast_to(packed, (_C, _W)) >> csub) & 1).astype(jnp.float32)

    def expand_y(packed):
        return ((jnp.broadcast_to(packed, (_H, _C)) >> clane) & 1).astype(jnp.float32)

    idx_x = lax.broadcasted_iota(jnp.int32, (1, _W), 1).astype(jnp.float32)
    anyx, x_min, x_max = _axis_stats(expand_x(scol), expand_x(pcol), 1, idx_x, _W)

    idx_y = lax.broadcasted_iota(jnp.int32, (_H, 1), 0).astype(jnp.float32)
    anyy_r, ylo_r, yhi_r = _axis_stats(expand_y(srow), expand_y(prow), 0, idx_y, _H)
    anyy = _diag_col(anyy_r)
    y_min = _diag_col(ylo_r)
    y_max = _diag_col(yhi_r)

    keep = anyx * anyy * jnp.where(csub != 0, 1.0, 0.0)
    bbox_ref[0] = jnp.concatenate([x_min, y_min, x_max, y_max], axis=1) * keep


@jax.jit
def _fuse(pcol, prow, scol, srow):
    return pl.pallas_call(
        _fuse_kernel,
        grid=(_B,),
        in_specs=[
            pl.BlockSpec((1, 1, _W), lambda b: (b, 0, 0)),
            pl.BlockSpec((1, _H, 1), lambda b: (b, 0, 0)),
            pl.BlockSpec((1, _NBAND, _W), lambda b: (b, 0, 0)),
            pl.BlockSpec((1, _H, 1), lambda b: (b, 0, 0)),
        ],
        out_specs=pl.BlockSpec((1, _C, 4), lambda b: (b, 0, 0)),
        out_shape=jax.ShapeDtypeStruct((_B, _C, 4), jnp.float32),
        compiler_params=pltpu.CompilerParams(
            dimension_semantics=("arbitrary",),
        ),
    )(pcol, prow, scol, srow)


def kernel(scribbles, outputs):
    scr = scribbles.astype(jnp.int32)
    scol, srow = _sc_proj(scr)
    pcol, prow = _pred_proj(outputs)
    bbox = _fuse(pcol, prow, scol.reshape(_B, _NBAND, _W),
                 srow.reshape(_B, _H, 1))
    return bbox
